# baseline (speedup 1.0000x reference)
import functools

import jax
import jax.numpy as jnp
from jax import lax
from jax.experimental import pallas as pl
from jax.experimental.pallas import tpu as pltpu
from jax.experimental.pallas import tpu_sc as plsc


def _dynamic_gather(x):
    B, C, H, W = x.shape
    R = H * W
    N = B * C
    xf = x.reshape(N, R)
    inds = jnp.linspace(C - 1, 0, C).astype(jnp.int32)
    src = (jnp.arange(B, dtype=jnp.int32)[:, None] * C + inds[None, :]).reshape(N)

    info = plsc.get_sparse_core_info()
    NC, NS = info.num_cores, info.num_subcores
    NW = NC * NS
    rows_per_w = N // NW
    SIZES = (16, 8)
    OFFS = (0, 16)
    PER = sum(SIZES)
    n_groups = rows_per_w // PER

    mesh = plsc.VectorSubcoreMesh(
        core_axis_name="c", subcore_axis_name="s", num_cores=NC
    )

    @functools.partial(
        pl.kernel,
        mesh=mesh,
        out_type=jax.ShapeDtypeStruct((N, R), jnp.float32),
        scratch_types=[
            pltpu.VMEM((rows_per_w,), jnp.int32),
            *[pltpu.VMEM((sz, R), jnp.float32) for sz in SIZES],
            *[pltpu.SemaphoreType.DMA for _ in range(2 * len(SIZES))],
        ],
    )
    def gather_rows(x_hbm, src_hbm, out_hbm, idx_v, *scratch):
        nb = len(SIZES)
        bufs = scratch[:nb]
        sg = scratch[nb : 2 * nb]
        ss = scratch[2 * nb :]
        wid = lax.axis_index("s") * NC + lax.axis_index("c")
        base = wid * rows_per_w
        pltpu.sync_copy(src_hbm.at[pl.ds(base, rows_per_w)], idx_v)

        def group(g, carry):
            for b in range(nb):
                sz = SIZES[b]
                off = g * PER + OFFS[b]
                pltpu.make_async_copy(
                    x_hbm.at[idx_v.at[pl.ds(off, sz)]], bufs[b], sg[b]
                ).wait()
                pltpu.async_copy(bufs[b], out_hbm.at[pl.ds(base + off, sz)], ss[b])

                @pl.when(g + 1 < n_groups)
                def _():
                    pltpu.make_async_copy(
                        bufs[b], out_hbm.at[pl.ds(base + off, sz)], ss[b]
                    ).wait()
                    pltpu.async_copy(
                        x_hbm.at[idx_v.at[pl.ds(off + PER, sz)]], bufs[b], sg[b]
                    )

            return carry

        for b in range(nb):
            pltpu.async_copy(
                x_hbm.at[idx_v.at[pl.ds(OFFS[b], SIZES[b])]], bufs[b], sg[b]
            )
        lax.fori_loop(0, n_groups, group, 0)
        for b in range(nb):
            pltpu.make_async_copy(
                bufs[b], out_hbm.at[pl.ds(base, SIZES[b])], ss[b]
            ).wait()

    return gather_rows(xf, src).reshape(B, C, H, W)


def kernel(x):
    return _dynamic_gather(x)


# speedup vs baseline: 1.0048x; 1.0048x over previous
import functools

import jax
import jax.numpy as jnp
from jax import lax
from jax.experimental import pallas as pl
from jax.experimental.pallas import tpu as pltpu
from jax.experimental.pallas import tpu_sc as plsc


def _dynamic_gather(x):
    B, C, H, W = x.shape
    R = H * W
    N = B * C
    xf = x.reshape(N, R)
    inds = jnp.linspace(C - 1, 0, C).astype(jnp.int32)
    src = (jnp.arange(B, dtype=jnp.int32)[:, None] * C + inds[None, :]).reshape(N)

    info = plsc.get_sparse_core_info()
    NC, NS = info.num_cores, info.num_subcores
    NW = NC * NS
    rows_per_w = N // NW
    G = 8
    NBUF = 3
    n_chunks = rows_per_w // G
    n_groups = n_chunks // NBUF

    mesh = plsc.VectorSubcoreMesh(
        core_axis_name="c", subcore_axis_name="s", num_cores=NC
    )

    @functools.partial(
        pl.kernel,
        mesh=mesh,
        out_type=jax.ShapeDtypeStruct((N, R), jnp.float32),
        scratch_types=[
            pltpu.VMEM((rows_per_w,), jnp.int32),
            *[pltpu.VMEM((G, R), jnp.float32) for _ in range(NBUF)],
            *[pltpu.SemaphoreType.DMA for _ in range(2 * NBUF)],
        ],
    )
    def gather_rows(x_hbm, src_hbm, out_hbm, idx_v, *scratch):
        bufs = scratch[:NBUF]
        sg = scratch[NBUF : 2 * NBUF]
        ss = scratch[2 * NBUF :]
        wid = lax.axis_index("s") * NC + lax.axis_index("c")
        base = wid * rows_per_w
        pltpu.sync_copy(src_hbm.at[pl.ds(base, rows_per_w)], idx_v)

        def group(g, carry):
            for b in range(NBUF):
                m = g * NBUF + b
                pltpu.make_async_copy(
                    x_hbm.at[idx_v.at[pl.ds(m * G, G)]], bufs[b], sg[b]
                ).wait()
                pltpu.async_copy(
                    bufs[b], out_hbm.at[pl.ds(base + m * G, G)], ss[b]
                )

                @pl.when(m + NBUF < n_chunks)
                def _():
                    pltpu.make_async_copy(
                        bufs[b], out_hbm.at[pl.ds(base + m * G, G)], ss[b]
                    ).wait()
                    pltpu.async_copy(
                        x_hbm.at[idx_v.at[pl.ds((m + NBUF) * G, G)]],
                        bufs[b],
                        sg[b],
                    )

            return carry

        for b in range(NBUF):
            pltpu.async_copy(x_hbm.at[idx_v.at[pl.ds(b * G, G)]], bufs[b], sg[b])
        lax.fori_loop(0, n_groups, group, 0)
        for b in range(NBUF):
            pltpu.make_async_copy(
                bufs[b], out_hbm.at[pl.ds(base, G)], ss[b]
            ).wait()

    return gather_rows(xf, src).reshape(B, C, H, W)


def kernel(x):
    return _dynamic_gather(x)
